# baseline (speedup 1.0000x reference)
import functools

import jax
import jax.numpy as jnp
from jax import lax
from jax.experimental import pallas as pl
from jax.experimental.pallas import tpu as pltpu
from jax.experimental.pallas import tpu_sc as plsc

_N = 10000
_E = 320000
_H = 128
_G = 64
_C = 10

_NC = 2
_NS = 16
_NW = _NC * _NS
_EPW = _E // _NW
_B = 80
_NCHUNK = _EPW // _B
_NP = 10240
_RPS = _NP // _NS
_K = 8

_mesh = plsc.VectorSubcoreMesh(core_axis_name="c", subcore_axis_name="s")


def _zero_fill(buf, rows, width):
    zero16 = jnp.zeros((16,), jnp.float32)

    def body(r, carry):
        for c in range(width // 16):
            buf[r, pl.ds(c * 16, 16)] = zero16
        return carry

    lax.fori_loop(0, rows, body, 0)


@functools.partial(
    pl.kernel,
    out_type=jax.ShapeDtypeStruct((_NC, _NP, 16), jnp.float32),
    mesh=_mesh,
    compiler_params=pltpu.CompilerParams(use_tc_tiling_on_sc=False),
    scratch_types=[
        pltpu.VMEM((_NCHUNK, 1, _B), jnp.int32),
        pltpu.VMEM((_B, 16), jnp.float32),
        pltpu.VMEM((_RPS, 16), jnp.float32),
        pltpu.VMEM_SHARED((_NP, 16), jnp.float32),
        pltpu.SemaphoreType.DMA,
        pltpu.SemaphoreType.DMA,
    ],
)
def _deg_kernel(dst3_hbm, out_hbm, didx3, ones_v, zbuf, hist, ssem, isem):
    cid = lax.axis_index("c")
    sid = lax.axis_index("s")
    wid = sid * _NC + cid
    idma = pltpu.async_copy(dst3_hbm.at[wid], didx3, isem)

    one16 = jnp.ones((16,), jnp.float32)

    def fill_ones(r, carry):
        ones_v[r, pl.ds(0, 16)] = one16
        return carry

    lax.fori_loop(0, _B, fill_ones, 0)
    _zero_fill(zbuf, _RPS, 16)
    pltpu.sync_copy(zbuf, hist.at[pl.ds(sid * _RPS, _RPS)])
    plsc.subcore_barrier()
    idma.wait()

    def fire(j):
        pltpu.async_copy(ones_v, hist.at[didx3.at[j, 0]], ssem, add=True)

    def drain():
        pltpu.make_async_copy(ones_v, hist.at[pl.ds(0, _B)], ssem).wait()

    for j in range(_K):
        fire(j)

    def body(j, carry):
        drain()
        fire(j + _K)
        return carry

    lax.fori_loop(0, _NCHUNK - _K, body, 0)
    for _ in range(_K):
        drain()
    plsc.subcore_barrier()
    pltpu.sync_copy(
        hist.at[pl.ds(sid * _RPS, _RPS)],
        out_hbm.at[cid, pl.ds(sid * _RPS, _RPS)],
    )


@functools.partial(
    pl.kernel,
    out_type=jax.ShapeDtypeStruct((_NC, _NP, _H), jnp.float32),
    mesh=_mesh,
    scratch_types=(
        [pltpu.VMEM((_EPW,), jnp.int32)]
        + [pltpu.VMEM((1, _B), jnp.int32) for _ in range(3)]
        + [pltpu.VMEM((_B, _H), jnp.float32) for _ in range(3)]
        + [pltpu.VMEM_SHARED((_NP, _H), jnp.float32)]
        + [pltpu.SemaphoreType.DMA] * 10
    ),
)
def _msg_kernel(src2_hbm, dst3_hbm, y_hbm, out_hbm,
                sidx_all, d0, d1, d2, r0, r1, r2, acc,
                gs0, gs1, gs2, ss0, ss1, ss2, ds0, ds1, ds2, isem):
    cid = lax.axis_index("c")
    sid = lax.axis_index("s")
    wid = sid * _NC + cid
    ia = pltpu.async_copy(src2_hbm.at[wid], sidx_all, isem)
    didx = [d0, d1, d2]
    rows = [r0, r1, r2]
    gsem = [gs0, gs1, gs2]
    ssem = [ss0, ss1, ss2]
    dsem = [ds0, ds1, ds2]

    _zero_fill(r0, _B, _H)
    for k in range(_RPS // _B):
        pltpu.sync_copy(r0, acc.at[pl.ds(sid * _RPS + k * _B, _B)])
    plsc.subcore_barrier()
    ia.wait()

    def fire_didx(j, b):
        pltpu.async_copy(dst3_hbm.at[wid, j], didx[b], dsem[b])

    def wait_didx(b):
        pltpu.make_async_copy(dst3_hbm.at[0, 0], didx[b], dsem[b]).wait()

    def fire_gather(j, b):
        for h in range(2):
            pltpu.async_copy(
                y_hbm.at[sidx_all.at[pl.ds(j * _B + h * (_B // 2), _B // 2)]],
                rows[b].at[pl.ds(h * (_B // 2), _B // 2)], gsem[b])

    def wait_gather(b):
        pltpu.make_async_copy(y_hbm.at[pl.ds(0, _B)], rows[b], gsem[b]).wait()

    def fire_scatter(j, b):
        pltpu.async_copy(rows[b], acc.at[didx[b].at[0]], ssem[b], add=True)

    def wait_scatter(b):
        pltpu.make_async_copy(rows[b], acc.at[pl.ds(0, _B)], ssem[b]).wait()

    for j in range(3):
        fire_didx(j, j)
        fire_gather(j, j)

    wait_gather(0); wait_didx(0); fire_scatter(0, 0)
    wait_gather(1); wait_didx(1); fire_scatter(1, 1)
    wait_scatter(0); fire_didx(3, 0); fire_gather(3, 0)
    wait_gather(2); wait_didx(2); fire_scatter(2, 2)
    wait_scatter(1); fire_didx(4, 1); fire_gather(4, 1)

    def steady(j3, carry):
        for b in range(3):
            j = 3 + 3 * j3 + b
            wait_gather(b)
            wait_didx(b)
            fire_scatter(j, b)
            ps = (b + 2) % 3
            wait_scatter(ps)
            fire_didx(j + 2, ps)
            fire_gather(j + 2, ps)
        return carry

    lax.fori_loop(0, (_NCHUNK - 5) // 3, steady, 0)

    wait_gather(0); wait_didx(0); fire_scatter(_NCHUNK - 2, 0)
    wait_gather(1); wait_didx(1); fire_scatter(_NCHUNK - 1, 1)
    wait_scatter(2)
    wait_scatter(0)
    wait_scatter(1)

    plsc.subcore_barrier()
    pltpu.sync_copy(
        acc.at[pl.ds(sid * _RPS, _RPS)],
        out_hbm.at[cid, pl.ds(sid * _RPS, _RPS)],
    )


def _tc_prep_body(hist_ref, x_ref, w1_ref, y1_ref, dis_ref):
    deg = 1.0 + hist_ref[0, 0:_N, 0:1] + hist_ref[1, 0:_N, 0:1]
    dis = lax.rsqrt(deg)
    xw = jnp.dot(x_ref[...], w1_ref[...], preferred_element_type=jnp.float32,
                 precision=lax.Precision.DEFAULT)
    y1_ref[...] = dis * xw
    dis_ref[...] = dis


_tc_prep = pl.pallas_call(
    _tc_prep_body,
    out_shape=(
        jax.ShapeDtypeStruct((_N, _H), jnp.float32),
        jax.ShapeDtypeStruct((_N, 1), jnp.float32),
    ),
)


def _tc_mid_body(p_ref, y1_ref, dis_ref, b1_ref, w2_ref, y2_ref):
    dis = dis_ref[...]
    h1 = jnp.maximum(
        dis * (p_ref[0, 0:_N, :] + p_ref[1, 0:_N, :] + y1_ref[...]) + b1_ref[...],
        0.0)
    hw = jnp.dot(h1, w2_ref[...], preferred_element_type=jnp.float32,
                 precision=lax.Precision.DEFAULT)
    y2_ref[...] = dis * hw


_tc_mid = pl.pallas_call(
    _tc_mid_body,
    out_shape=jax.ShapeDtypeStruct((_N, _H), jnp.float32),
)


def _tc_final_body(p_ref, y2_ref, dis_ref, b2_ref, batch_ref, wfc_ref, bfc_ref,
                   out_ref):
    h2 = (dis_ref[...] * (p_ref[0, 0:_N, :] + p_ref[1, 0:_N, :] + y2_ref[...])
          + b2_ref[...])
    gids = lax.broadcasted_iota(jnp.int32, (_G, _N), 0)
    onehot = (batch_ref[...] == gids).astype(jnp.float32)
    sums = jnp.dot(onehot, h2, preferred_element_type=jnp.float32,
                   precision=lax.Precision.DEFAULT)
    counts = jnp.sum(onehot, axis=1, keepdims=True)
    pooled = sums / jnp.maximum(counts, 1.0)
    out_ref[...] = jnp.dot(pooled, wfc_ref[...], preferred_element_type=jnp.float32,
                           precision=lax.Precision.DEFAULT) + bfc_ref[...]


_tc_final = pl.pallas_call(
    _tc_final_body,
    out_shape=jax.ShapeDtypeStruct((_G, _C), jnp.float32),
)


@jax.jit
def kernel(x, edge_index, batch, W1, b1, W2, b2, Wfc, bfc):
    src2 = edge_index[0].reshape(_NW, _EPW)
    dst3 = edge_index[1].reshape(_NW, _NCHUNK, 1, _B)
    hist = _deg_kernel(dst3)
    y1, dis = _tc_prep(hist, x, W1)
    p1 = _msg_kernel(src2, dst3, y1)
    y2 = _tc_mid(p1, y1, dis, b1.reshape(1, _H), W2)
    p2 = _msg_kernel(src2, dst3, y2)
    return _tc_final(p2, y2, dis, b2.reshape(1, _H), batch.reshape(1, _N),
                     Wfc, bfc.reshape(1, _C))


# speedup vs baseline: 1.0014x; 1.0014x over previous
import functools

import jax
import jax.numpy as jnp
from jax import lax
from jax.experimental import pallas as pl
from jax.experimental.pallas import tpu as pltpu
from jax.experimental.pallas import tpu_sc as plsc

_N = 10000
_E = 320000
_H = 128
_G = 64
_C = 10

_NC = 2
_NS = 16
_NW = _NC * _NS
_EPW = _E // _NW
_B = 80
_NCHUNK = _EPW // _B
_NP = 10240
_RPS = _NP // _NS
_K = 8

_mesh = plsc.VectorSubcoreMesh(core_axis_name="c", subcore_axis_name="s")


def _zero_fill(buf, rows, width):
    zero16 = jnp.zeros((16,), jnp.float32)

    def body(r, carry):
        for c in range(width // 16):
            buf[r, pl.ds(c * 16, 16)] = zero16
        return carry

    lax.fori_loop(0, rows, body, 0)


@functools.partial(
    pl.kernel,
    out_type=jax.ShapeDtypeStruct((_NC, _NP, 16), jnp.float32),
    mesh=_mesh,
    compiler_params=pltpu.CompilerParams(use_tc_tiling_on_sc=False),
    scratch_types=[
        pltpu.VMEM((_NCHUNK, 1, _B), jnp.int32),
        pltpu.VMEM((_B, 16), jnp.float32),
        pltpu.VMEM((_RPS, 16), jnp.float32),
        pltpu.VMEM_SHARED((_NP, 16), jnp.float32),
        pltpu.SemaphoreType.DMA,
        pltpu.SemaphoreType.DMA,
    ],
)
def _deg_kernel(dst3_hbm, out_hbm, didx3, ones_v, zbuf, hist, ssem, isem):
    cid = lax.axis_index("c")
    sid = lax.axis_index("s")
    wid = sid * _NC + cid
    idma = pltpu.async_copy(dst3_hbm.at[wid], didx3, isem)

    one16 = jnp.ones((16,), jnp.float32)

    def fill_ones(r, carry):
        ones_v[r, pl.ds(0, 16)] = one16
        return carry

    lax.fori_loop(0, _B, fill_ones, 0)
    _zero_fill(zbuf, _RPS, 16)
    pltpu.sync_copy(zbuf, hist.at[pl.ds(sid * _RPS, _RPS)])
    plsc.subcore_barrier()
    idma.wait()

    def fire(j):
        pltpu.async_copy(ones_v, hist.at[didx3.at[j, 0]], ssem, add=True)

    def drain():
        pltpu.make_async_copy(ones_v, hist.at[pl.ds(0, _B)], ssem).wait()

    for j in range(_K):
        fire(j)

    def body(j, carry):
        drain()
        fire(j + _K)
        return carry

    lax.fori_loop(0, _NCHUNK - _K, body, 0)
    for _ in range(_K):
        drain()
    plsc.subcore_barrier()
    pltpu.sync_copy(
        hist.at[pl.ds(sid * _RPS, _RPS)],
        out_hbm.at[cid, pl.ds(sid * _RPS, _RPS)],
    )


@functools.partial(
    pl.kernel,
    out_type=jax.ShapeDtypeStruct((_NC, _NP, _H), jnp.float32),
    mesh=_mesh,
    scratch_types=(
        [pltpu.VMEM((_EPW,), jnp.int32)]
        + [pltpu.VMEM((1, _B), jnp.int32) for _ in range(3)]
        + [pltpu.VMEM((_B, _H), jnp.float32) for _ in range(3)]
        + [pltpu.VMEM_SHARED((_NP, _H), jnp.float32)]
        + [pltpu.SemaphoreType.DMA] * 10
    ),
)
def _msg_kernel(src2_hbm, dst3_hbm, y_hbm, out_hbm,
                sidx_all, d0, d1, d2, r0, r1, r2, acc,
                gs0, gs1, gs2, ss0, ss1, ss2, ds0, ds1, ds2, isem):
    cid = lax.axis_index("c")
    sid = lax.axis_index("s")
    wid = sid * _NC + cid
    ia = pltpu.async_copy(src2_hbm.at[wid], sidx_all, isem)
    didx = [d0, d1, d2]
    rows = [r0, r1, r2]
    gsem = [gs0, gs1, gs2]
    ssem = [ss0, ss1, ss2]
    dsem = [ds0, ds1, ds2]

    _zero_fill(r0, _B, _H)
    for k in range(_RPS // _B):
        pltpu.sync_copy(r0, acc.at[pl.ds(sid * _RPS + k * _B, _B)])
    plsc.subcore_barrier()
    ia.wait()

    def fire_didx(j, b):
        pltpu.async_copy(dst3_hbm.at[wid, j], didx[b], dsem[b])

    def wait_didx(b):
        pltpu.make_async_copy(dst3_hbm.at[0, 0], didx[b], dsem[b]).wait()

    def fire_gather(j, b):
        pltpu.async_copy(y_hbm.at[sidx_all.at[pl.ds(j * _B, _B)]], rows[b], gsem[b])

    def wait_gather(b):
        pltpu.make_async_copy(y_hbm.at[pl.ds(0, _B)], rows[b], gsem[b]).wait()

    def fire_scatter(j, b):
        pltpu.async_copy(rows[b], acc.at[didx[b].at[0]], ssem[b], add=True)

    def wait_scatter(b):
        pltpu.make_async_copy(rows[b], acc.at[pl.ds(0, _B)], ssem[b]).wait()

    for j in range(3):
        fire_didx(j, j)
        fire_gather(j, j)

    wait_gather(0); wait_didx(0); fire_scatter(0, 0)
    wait_gather(1); wait_didx(1); fire_scatter(1, 1)
    wait_scatter(0); fire_didx(3, 0); fire_gather(3, 0)
    wait_gather(2); wait_didx(2); fire_scatter(2, 2)
    wait_scatter(1); fire_didx(4, 1); fire_gather(4, 1)

    def steady(j3, carry):
        for b in range(3):
            j = 3 + 3 * j3 + b
            wait_gather(b)
            wait_didx(b)
            fire_scatter(j, b)
            ps = (b + 2) % 3
            wait_scatter(ps)
            fire_didx(j + 2, ps)
            fire_gather(j + 2, ps)
        return carry

    lax.fori_loop(0, (_NCHUNK - 5) // 3, steady, 0)

    wait_gather(0); wait_didx(0); fire_scatter(_NCHUNK - 2, 0)
    wait_gather(1); wait_didx(1); fire_scatter(_NCHUNK - 1, 1)
    wait_scatter(2)
    wait_scatter(0)
    wait_scatter(1)

    plsc.subcore_barrier()
    pltpu.sync_copy(
        acc.at[pl.ds(sid * _RPS, _RPS)],
        out_hbm.at[cid, pl.ds(sid * _RPS, _RPS)],
    )


def _tc_prep_body(hist_ref, x_ref, w1_ref, y1_ref, dis_ref):
    deg = 1.0 + hist_ref[0, 0:_N, 0:1] + hist_ref[1, 0:_N, 0:1]
    dis = lax.rsqrt(deg)
    xw = jnp.dot(x_ref[...], w1_ref[...], preferred_element_type=jnp.float32,
                 precision=lax.Precision.DEFAULT)
    y1_ref[...] = dis * xw
    dis_ref[...] = dis


_tc_prep = pl.pallas_call(
    _tc_prep_body,
    out_shape=(
        jax.ShapeDtypeStruct((_N, _H), jnp.float32),
        jax.ShapeDtypeStruct((_N, 1), jnp.float32),
    ),
)


def _tc_mid_body(p_ref, y1_ref, dis_ref, b1_ref, w2_ref, y2_ref):
    dis = dis_ref[...]
    h1 = jnp.maximum(
        dis * (p_ref[0, 0:_N, :] + p_ref[1, 0:_N, :] + y1_ref[...]) + b1_ref[...],
        0.0)
    hw = jnp.dot(h1, w2_ref[...], preferred_element_type=jnp.float32,
                 precision=lax.Precision.DEFAULT)
    y2_ref[...] = dis * hw


_tc_mid = pl.pallas_call(
    _tc_mid_body,
    out_shape=jax.ShapeDtypeStruct((_N, _H), jnp.float32),
)


def _tc_final_body(p_ref, y2_ref, dis_ref, b2_ref, batch_ref, wfc_ref, bfc_ref,
                   out_ref):
    h2 = (dis_ref[...] * (p_ref[0, 0:_N, :] + p_ref[1, 0:_N, :] + y2_ref[...])
          + b2_ref[...])
    gids = lax.broadcasted_iota(jnp.int32, (_G, _N), 0)
    onehot = (batch_ref[...] == gids).astype(jnp.float32)
    sums = jnp.dot(onehot, h2, preferred_element_type=jnp.float32,
                   precision=lax.Precision.DEFAULT)
    counts = jnp.sum(onehot, axis=1, keepdims=True)
    pooled = sums / jnp.maximum(counts, 1.0)
    out_ref[...] = jnp.dot(pooled, wfc_ref[...], preferred_element_type=jnp.float32,
                           precision=lax.Precision.DEFAULT) + bfc_ref[...]


_tc_final = pl.pallas_call(
    _tc_final_body,
    out_shape=jax.ShapeDtypeStruct((_G, _C), jnp.float32),
)


@jax.jit
def kernel(x, edge_index, batch, W1, b1, W2, b2, Wfc, bfc):
    src2 = edge_index[0].reshape(_NW, _EPW)
    dst3 = edge_index[1].reshape(_NW, _NCHUNK, 1, _B)
    hist = _deg_kernel(dst3)
    y1, dis = _tc_prep(hist, x, W1)
    p1 = _msg_kernel(src2, dst3, y1)
    y2 = _tc_mid(p1, y1, dis, b1.reshape(1, _H), W2)
    p2 = _msg_kernel(src2, dst3, y2)
    return _tc_final(p2, y2, dis, b2.reshape(1, _H), batch.reshape(1, _N),
                     Wfc, bfc.reshape(1, _C))
